# baseline (speedup 1.0000x reference)
import jax
import jax.numpy as jnp
from jax.experimental import pallas as pl
from jax.experimental.pallas import tpu as pltpu

NB = 16
D = 64
K = 1024
T = 1024
N = NB * T
COMMITMENT_COST = 0.25


def _vq_body(x_ref, e_ref, idx_ref, loss_ref, perp_ref,
             te_ref, e2_ref, iotaf_ref, hist_ref, msum_ref):
    i = pl.program_id(0)

    @pl.when(i == 0)
    def _init():
        msum_ref[0] = 0.0
        hist_ref[...] = jnp.zeros_like(hist_ref)
        e = e_ref[...]
        te_ref[...] = 2.0 * e
        e2_ref[...] = jnp.sum(e * e, axis=1, keepdims=True)
        iotaf_ref[...] = jax.lax.broadcasted_iota(
            jnp.int32, (K, T), 0).astype(jnp.float32)

    xt = x_ref[0].reshape(D, T)
    s2 = jax.lax.dot_general(te_ref[...], xt, (((1,), (0,)), ((), ())),
                             preferred_element_type=jnp.float32)
    x2 = jnp.sum(xt * xt, axis=0, keepdims=True)
    d = (x2 + e2_ref[...]) - s2
    mind = jnp.min(d, axis=0, keepdims=True)
    eq = d == mind
    idxf = jnp.min(jnp.where(eq, iotaf_ref[...], float(K)),
                   axis=0, keepdims=True)
    idx = idxf.astype(jnp.int32)
    idx_ref[0, 0, :] = idx[0]
    msum_ref[0] += jnp.sum(mind)
    hi = idx >> 5
    lo = idx & 31
    iota32 = jax.lax.broadcasted_iota(jnp.int32, (32, T), 0)
    oh_hi = (iota32 == hi).astype(jnp.float32)
    oh_lo = (iota32 == lo).astype(jnp.float32)
    hist_ref[...] += jax.lax.dot_general(
        oh_hi, oh_lo, (((1,), (1,)), ((), ())),
        preferred_element_type=jnp.float32)

    @pl.when(i == NB - 1)
    def _fin():
        loss = COMMITMENT_COST * (msum_ref[0] / (N * D))
        loss_ref[...] = jnp.broadcast_to(loss, (1, 1))
        p = hist_ref[...] / N
        perp = jnp.exp(-jnp.sum(p * jnp.log(p + 1e-10)))
        perp_ref[...] = jnp.broadcast_to(perp, (1, 1))


def kernel(inputs, embedding_weight):
    n, c, h, w = inputs.shape
    x = inputs

    idx, loss, perp = pl.pallas_call(
        _vq_body,
        grid=(NB,),
        in_specs=[
            pl.BlockSpec((1, D, h, w), lambda i: (i, 0, 0, 0)),
            pl.BlockSpec((K, D), lambda i: (0, 0)),
        ],
        out_specs=[
            pl.BlockSpec((1, 1, T), lambda i: (i, 0, 0)),
            pl.BlockSpec((1, 1), lambda i: (0, 0)),
            pl.BlockSpec((1, 1), lambda i: (0, 0)),
        ],
        out_shape=[
            jax.ShapeDtypeStruct((NB, 1, T), jnp.int32),
            jax.ShapeDtypeStruct((1, 1), jnp.float32),
            jax.ShapeDtypeStruct((1, 1), jnp.float32),
        ],
        scratch_shapes=[
            pltpu.VMEM((K, D), jnp.float32),
            pltpu.VMEM((K, 1), jnp.float32),
            pltpu.VMEM((K, T), jnp.float32),
            pltpu.VMEM((32, 32), jnp.float32),
            pltpu.SMEM((1,), jnp.float32),
        ],
    )(x, embedding_weight)

    return (loss[0, 0], perp[0, 0], idx.reshape(n, 1, h, w))


# speedup vs baseline: 1.3864x; 1.3864x over previous
import jax
import jax.numpy as jnp
from jax.experimental import pallas as pl
from jax.experimental.pallas import tpu as pltpu

NB = 16
BB = 2
D = 64
K = 1024
T = 1024
N = NB * T
COMMITMENT_COST = 0.25


def _vq_body(x_ref, e_ref, idx_ref, loss_ref, perp_ref,
             te_ref, e2_ref, iotaf_ref, hist_ref, msum_ref):
    i = pl.program_id(0)

    @pl.when(i == 0)
    def _init():
        msum_ref[0] = 0.0
        hist_ref[...] = jnp.zeros_like(hist_ref)
        e = e_ref[...]
        te_ref[...] = 2.0 * e
        e2_ref[...] = jnp.sum(e * e, axis=1, keepdims=True)
        iotaf_ref[...] = jax.lax.broadcasted_iota(
            jnp.int32, (K, T), 0).astype(jnp.float32)

    iota32 = jax.lax.broadcasted_iota(jnp.int32, (32, T), 0)
    for b in range(BB):
        xt = x_ref[b]
        s2 = jax.lax.dot_general(te_ref[...], xt, (((1,), (0,)), ((), ())),
                                 preferred_element_type=jnp.float32)
        x2 = jnp.sum(xt * xt, axis=0, keepdims=True)
        d = (x2 + e2_ref[...]) - s2
        mind = jnp.min(d, axis=0, keepdims=True)
        eq = d == mind
        idxf = jnp.min(jnp.where(eq, iotaf_ref[...], float(K)),
                       axis=0, keepdims=True)
        idx = idxf.astype(jnp.int32)
        idx_ref[b, 0, :] = idx[0]
        msum_ref[0] += jnp.sum(mind)
        hi = idx >> 5
        lo = idx & 31
        oh_hi = (iota32 == hi).astype(jnp.float32)
        oh_lo = (iota32 == lo).astype(jnp.float32)
        hist_ref[...] += jax.lax.dot_general(
            oh_hi, oh_lo, (((1,), (1,)), ((), ())),
            preferred_element_type=jnp.float32)

    @pl.when(i == NB // BB - 1)
    def _fin():
        loss = COMMITMENT_COST * (msum_ref[0] / (N * D))
        loss_ref[...] = jnp.broadcast_to(loss, (1, 1))
        p = hist_ref[...] / N
        perp = jnp.exp(-jnp.sum(p * jnp.log(p + 1e-10)))
        perp_ref[...] = jnp.broadcast_to(perp, (1, 1))


def kernel(inputs, embedding_weight):
    n, c, h, w = inputs.shape
    x = inputs.reshape(NB, D, T)

    idx, loss, perp = pl.pallas_call(
        _vq_body,
        grid=(NB // BB,),
        in_specs=[
            pl.BlockSpec((BB, D, T), lambda i: (i, 0, 0)),
            pl.BlockSpec((K, D), lambda i: (0, 0)),
        ],
        out_specs=[
            pl.BlockSpec((BB, 1, T), lambda i: (i, 0, 0)),
            pl.BlockSpec((1, 1), lambda i: (0, 0)),
            pl.BlockSpec((1, 1), lambda i: (0, 0)),
        ],
        out_shape=[
            jax.ShapeDtypeStruct((NB, 1, T), jnp.int32),
            jax.ShapeDtypeStruct((1, 1), jnp.float32),
            jax.ShapeDtypeStruct((1, 1), jnp.float32),
        ],
        scratch_shapes=[
            pltpu.VMEM((K, D), jnp.float32),
            pltpu.VMEM((K, 1), jnp.float32),
            pltpu.VMEM((K, T), jnp.float32),
            pltpu.VMEM((32, 32), jnp.float32),
            pltpu.SMEM((1,), jnp.float32),
        ],
    )(x, embedding_weight)

    return (loss[0, 0], perp[0, 0], idx.reshape(n, 1, h, w))
